# Initial kernel scaffold; baseline (speedup 1.0000x reference)
import jax
import jax.numpy as jnp
from jax.experimental import pallas as pl


def kernel(x0, x1, edge_index, edge_type, edge_label_index, lin0_w, lin0_b, lin1_w, lin1_b, bases1, comp1, root1, bias1, bases2, comp2, root2, bias2, fc_w, fc_b):
    raise NotImplementedError("write your pallas kernel here")


# speedup vs baseline: 4.7424x; 4.7424x over previous
import functools

import jax
import jax.numpy as jnp
from jax import lax
from jax.experimental import pallas as pl
from jax.experimental.pallas import tpu as pltpu
from jax.experimental.pallas import tpu_sc as plsc

N = 10000
D_IN = 128
D_HID = 128
D_OUT = 64
NREL = 4
E = 320000
E_LBL = 100000

NC, NS, L = 2, 16, 16
NW = NC * NS
CW = 128
CH = 80
WID = 16
NPASS = D_IN // WID
E_PAD = NW * CH * CW
ROWS = 40960
RPW = ROWS // NS
LBL_PW = 3136
E_LBL_PAD = NW * LBL_PW

_MESH = dict(core_axis_name="c", subcore_axis_name="s",
             num_cores=NC, num_subcores=NS)


def _fill_zeros(ref, nrows, width):
    z16 = jnp.zeros((16,), jnp.float32)

    def body(i, _):
        for m in range(width // 16):
            ref[i, pl.ds(m * 16, 16)] = z16
        return 0

    lax.fori_loop(0, nrows, body, 0, unroll=False)


def _sc_counts_body(dst3, typ3, kidx_out, cnt_out,
                    kidxv, rowbuf, onesb, zbufb, acc_cnt):
    cid = lax.axis_index("c")
    sid = lax.axis_index("s")
    wid = cid * NS + sid
    r0 = sid * RPW

    def p1(j, _):
        pltpu.sync_copy(dst3.at[wid, j], rowbuf.at[0])
        pltpu.sync_copy(typ3.at[wid, j], rowbuf.at[1])
        for m in range(CW // 16):
            d = rowbuf[0, pl.ds(m * 16, 16)]
            t = rowbuf[1, pl.ds(m * 16, 16)]
            kidxv[j, pl.ds(m * 16, 16)] = t * N + d
        return 0

    lax.fori_loop(0, CH, p1, 0, unroll=False)
    pltpu.sync_copy(kidxv, kidx_out.at[wid])

    def fill_ones(i, _):
        onesb[i] = jnp.ones((16,), jnp.float32)
        return 0

    lax.fori_loop(0, CW, fill_ones, 0, unroll=False)
    _fill_zeros(zbufb, 512, 16)
    for q in range(RPW // 512):
        pltpu.sync_copy(zbufb, acc_cnt.at[pl.ds(r0 + q * 512, 512)])
    plsc.subcore_barrier()

    def p2(j, _):
        pltpu.sync_copy(onesb, acc_cnt.at[kidxv.at[j]], add=True)
        return 0

    lax.fori_loop(0, CH, p2, 0, unroll=False)
    plsc.subcore_barrier()
    pltpu.sync_copy(acc_cnt.at[pl.ds(r0, RPW)],
                    cnt_out.at[cid, pl.ds(r0, RPW)])


def _sc_counts(dst3, typ3):
    return pl.kernel(
        _sc_counts_body,
        out_type=[
            jax.ShapeDtypeStruct((NW, CH, CW), jnp.int32),
            jax.ShapeDtypeStruct((NC, ROWS, 16), jnp.float32),
        ],
        mesh=plsc.VectorSubcoreMesh(**_MESH),
        compiler_params=pltpu.CompilerParams(use_tc_tiling_on_sc=False),
        scratch_types=[
            pltpu.VMEM((CH, CW), jnp.int32),
            pltpu.VMEM((2, CW), jnp.int32),
            pltpu.VMEM((CW, 16), jnp.float32),
            pltpu.VMEM((512, 16), jnp.float32),
            pltpu.VMEM_SHARED((ROWS, 16), jnp.float32),
        ],
    )(dst3, typ3)


def _sc_layer_body(*args):
    tables = list(args[:NPASS])
    (src3, kidx3, s_out, srcv, kidxv, zbufa, gbuf, sem, acc) = args[NPASS:]
    cid = lax.axis_index("c")
    sid = lax.axis_index("s")
    wid = cid * NS + sid
    r0 = sid * RPW

    pltpu.sync_copy(src3.at[wid], srcv)
    pltpu.sync_copy(kidx3.at[wid], kidxv)
    _fill_zeros(zbufa, RPW // 2, WID)

    for c, tc in enumerate(tables):
        pltpu.sync_copy(zbufa, acc.at[pl.ds(r0, RPW // 2)])
        pltpu.sync_copy(zbufa, acc.at[pl.ds(r0 + RPW // 2, RPW // 2)])
        plsc.subcore_barrier()

        def p3(j, _):
            pltpu.async_copy(tc.at[srcv.at[j]], gbuf, sem).wait()
            pltpu.sync_copy(gbuf, acc.at[kidxv.at[j]], add=True)
            return 0

        lax.fori_loop(0, CH, p3, 0, unroll=False)
        plsc.subcore_barrier()
        pltpu.sync_copy(acc.at[pl.ds(r0, RPW)],
                        s_out.at[cid, pl.ds(r0, RPW), pl.ds(c * WID, WID)])
        plsc.subcore_barrier()


def _sc_layer(tables, src3, kidx3):
    return pl.kernel(
        _sc_layer_body,
        out_type=[
            jax.ShapeDtypeStruct((NC, ROWS, D_IN), jnp.float32),
        ],
        mesh=plsc.VectorSubcoreMesh(**_MESH),
        compiler_params=pltpu.CompilerParams(use_tc_tiling_on_sc=False),
        scratch_types=[
            pltpu.VMEM((CH, CW), jnp.int32),
            pltpu.VMEM((CH, CW), jnp.int32),
            pltpu.VMEM((RPW // 2, WID), jnp.float32),
            pltpu.VMEM((CW, WID), jnp.float32),
            pltpu.SemaphoreType.DMA,
            pltpu.VMEM_SHARED((ROWS, WID), jnp.float32),
        ],
    )(*tables, src3, kidx3)[0]


def _sc_decode_body(scores, ls, ld, out, sc_v, lsv, ldv, ob):
    cid = lax.axis_index("c")
    sid = lax.axis_index("s")
    wid = cid * NS + sid
    pltpu.sync_copy(scores, sc_v)
    pltpu.sync_copy(ls.at[wid], lsv)
    pltpu.sync_copy(ld.at[wid], ldv)

    def body(i, _):
        s = lsv[pl.ds(i * 16, 16)]
        d = ldv[pl.ds(i * 16, 16)]
        a = plsc.load_gather(sc_v, [s * 2])
        b = plsc.load_gather(sc_v, [d * 2 + 1])
        ob[pl.ds(i * 16, 16)] = a + b
        return 0

    lax.fori_loop(0, LBL_PW // 16, body, 0, unroll=False)
    pltpu.sync_copy(ob, out.at[wid])


def _sc_decode(scores, ls, ld):
    return pl.kernel(
        _sc_decode_body,
        out_type=[jax.ShapeDtypeStruct((NW, LBL_PW), jnp.float32)],
        mesh=plsc.VectorSubcoreMesh(**_MESH),
        compiler_params=pltpu.CompilerParams(use_tc_tiling_on_sc=False,
                                             needs_layout_passes=False),
        scratch_types=[
            pltpu.VMEM((2 * N,), jnp.float32),
            pltpu.VMEM((LBL_PW,), jnp.int32),
            pltpu.VMEM((LBL_PW,), jnp.int32),
            pltpu.VMEM((LBL_PW,), jnp.float32),
        ],
    )(scores, ls, ld)[0]


_BLK = 1000
_NBLK = N // _BLK


def _proj_body(x0_ref, x1_ref, w0_ref, b0_ref, w1_ref, b1_ref,
               xc_ref, *chunk_refs):
    i = pl.program_id(0)
    y0 = jnp.dot(x0_ref[...], w0_ref[...],
                 preferred_element_type=jnp.float32) + b0_ref[...]
    y1 = jnp.dot(x1_ref[...], w1_ref[...],
                 preferred_element_type=jnp.float32) + b1_ref[...]
    y = jnp.where(i < _NBLK // 2, y0, y1)
    xc_ref[...] = y
    for c, ref in enumerate(chunk_refs):
        ref[...] = y[:, c * WID:(c + 1) * WID]


def _proj(x0, x1, w0, b0, w1, b1):
    half = _NBLK // 2
    chunk_spec = pl.BlockSpec((_BLK, WID), lambda i: (i, 0))
    return pl.pallas_call(
        _proj_body,
        grid=(_NBLK,),
        in_specs=[
            pl.BlockSpec((_BLK, D_IN), lambda i: (jnp.minimum(i, half - 1), 0)),
            pl.BlockSpec((_BLK, D_IN), lambda i: (jnp.maximum(i - half, 0), 0)),
            pl.BlockSpec((D_IN, D_IN), lambda i: (0, 0)),
            pl.BlockSpec((1, D_IN), lambda i: (0, 0)),
            pl.BlockSpec((D_IN, D_IN), lambda i: (0, 0)),
            pl.BlockSpec((1, D_IN), lambda i: (0, 0)),
        ],
        out_specs=[
            pl.BlockSpec((_BLK, D_IN), lambda i: (i, 0)),
        ] + [chunk_spec] * NPASS,
        out_shape=[
            jax.ShapeDtypeStruct((N, D_IN), jnp.float32),
        ] + [jax.ShapeDtypeStruct((N, WID), jnp.float32)] * NPASS,
    )(x0, x1, w0, b0, w1, b1)


def _wbuild_body(c1_ref, b1_ref, c2_ref, b2_ref, w1_ref, w2_ref):
    w1_ref[...] = jnp.dot(c1_ref[...], b1_ref[...],
                          preferred_element_type=jnp.float32)
    w2_ref[...] = jnp.dot(c2_ref[...], b2_ref[...],
                          preferred_element_type=jnp.float32)


def _wbuild(comp1, b1f, comp2, b2f):
    return pl.pallas_call(
        _wbuild_body,
        out_shape=[
            jax.ShapeDtypeStruct((NREL, D_IN * D_HID), jnp.float32),
            jax.ShapeDtypeStruct((NREL, D_HID * D_OUT), jnp.float32),
        ],
    )(comp1, b1f, comp2, b2f)


def _layer_tc_body(dout, last, x_ref, s0, s1, s2, s3, c0, c1, c2, c3,
                   w_ref, root_ref, bias_ref, *out_refs):
    acc = jnp.dot(x_ref[...], root_ref[...],
                  preferred_element_type=jnp.float32) + bias_ref[...]
    for r, (s_ref, c_ref) in enumerate(zip((s0, s1, s2, s3),
                                           (c0, c1, c2, c3))):
        h = s_ref[0] + s_ref[1]
        cnt = c_ref[0, :, 0] + c_ref[1, :, 0]
        inv = 1.0 / jnp.maximum(cnt, 1.0)
        h = h * inv[:, None]
        acc = acc + jnp.dot(h, w_ref[r], preferred_element_type=jnp.float32)
    if not last:
        y = jnp.maximum(acc, 0.0)
        out_refs[0][...] = y
        for c in range(NPASS):
            out_refs[1 + c][...] = y[:, c * WID:(c + 1) * WID]
    else:
        fct_ref, fcb_ref, scores_ref = out_refs
        scores = jnp.dot(acc, fct_ref[...],
                         preferred_element_type=jnp.float32) + fcb_ref[...]
        scores_ref[...] = scores


def _layer_tc(x, s, cnt, w, root, bias, last, fct=None, fcb=None):
    dout = D_OUT if last else D_HID

    def s_spec(r):
        return pl.BlockSpec((NC, _BLK, D_IN),
                            lambda i, r=r: (0, r * _NBLK + i, 0))

    def c_spec(r):
        return pl.BlockSpec((NC, _BLK, 16),
                            lambda i, r=r: (0, r * _NBLK + i, 0))

    in_specs = [pl.BlockSpec((_BLK, x.shape[1]), lambda i: (i, 0))]
    in_specs += [s_spec(r) for r in range(NREL)]
    in_specs += [c_spec(r) for r in range(NREL)]
    in_specs += [
        pl.BlockSpec((NREL, w.shape[1], dout), lambda i: (0, 0, 0)),
        pl.BlockSpec((root.shape[0], dout), lambda i: (0, 0)),
        pl.BlockSpec((1, dout), lambda i: (0, 0)),
    ]
    args = [x, s, s, s, s, cnt, cnt, cnt, cnt, w, root, bias]
    if not last:
        out_specs = [pl.BlockSpec((_BLK, dout), lambda i: (i, 0))]
        out_specs += [pl.BlockSpec((_BLK, WID), lambda i: (i, 0))] * NPASS
        out_shape = [jax.ShapeDtypeStruct((N, dout), jnp.float32)]
        out_shape += [jax.ShapeDtypeStruct((N, WID), jnp.float32)] * NPASS
    else:
        in_specs += [
            pl.BlockSpec((D_OUT, 2), lambda i: (0, 0)),
            pl.BlockSpec((1, 2), lambda i: (0, 0)),
        ]
        args += [fct, fcb]
        out_specs = [pl.BlockSpec((_BLK, 2), lambda i: (i, 0))]
        out_shape = [jax.ShapeDtypeStruct((N, 2), jnp.float32)]
    return pl.pallas_call(
        functools.partial(_layer_tc_body, dout, last),
        grid=(_NBLK,),
        in_specs=in_specs,
        out_specs=out_specs,
        out_shape=out_shape,
    )(*args)


def kernel(x0, x1, edge_index, edge_type, edge_label_index,
           lin0_w, lin0_b, lin1_w, lin1_b,
           bases1, comp1, root1, bias1,
           bases2, comp2, root2, bias2,
           fc_w, fc_b):
    f32 = jnp.float32
    src = edge_index[0]
    dst = edge_index[1]
    pad = E_PAD - E
    src3 = jnp.pad(src, (0, pad)).reshape(NW, CH, CW)
    dst3 = jnp.pad(dst, (0, pad)).reshape(NW, CH, CW)
    typ3 = jnp.pad(edge_type, (0, pad), constant_values=NREL).reshape(NW, CH, CW)
    ls = jnp.pad(edge_label_index[0], (0, E_LBL_PAD - E_LBL)).reshape(NW, LBL_PW)
    ld = jnp.pad(edge_label_index[1], (0, E_LBL_PAD - E_LBL)).reshape(NW, LBL_PW)

    xcat, *xchunks = _proj(
        x0, x1, lin0_w, lin0_b.reshape(1, -1), lin1_w, lin1_b.reshape(1, -1))
    w1f, w2f = _wbuild(comp1, bases1.reshape(30, -1), comp2,
                       bases2.reshape(30, -1))
    w1 = w1f.reshape(NREL, D_IN, D_HID)
    w2 = w2f.reshape(NREL, D_HID, D_OUT)

    kidx3, cnt = _sc_counts(dst3, typ3)
    s1 = _sc_layer(xchunks, src3, kidx3)
    z, *zchunks = _layer_tc(
        xcat, s1, cnt, w1, root1, bias1.reshape(1, -1), last=False)
    s2 = _sc_layer(zchunks, src3, kidx3)

    fct = jnp.stack([fc_w[:D_OUT, 0], fc_w[D_OUT:, 0]], axis=1)
    fcb = jnp.concatenate([fc_b, jnp.zeros((1,), f32)]).reshape(1, 2)
    (scores,) = _layer_tc(z, s2, cnt, w2, root2, bias2.reshape(1, -1),
                          last=True, fct=fct, fcb=fcb)

    out = _sc_decode(scores.reshape(-1), ls, ld)
    return out.reshape(-1)[:E_LBL].reshape(E_LBL, 1)


# speedup vs baseline: 6.8902x; 1.4529x over previous
import functools

import jax
import jax.numpy as jnp
from jax import lax
from jax.experimental import pallas as pl
from jax.experimental.pallas import tpu as pltpu
from jax.experimental.pallas import tpu_sc as plsc

N = 10000
D_IN = 128
D_HID = 128
D_OUT = 64
NREL = 4
E = 320000
E_LBL = 100000

NC, NS, L = 2, 16, 16
NW = NC * NS
CW = 128
CH = 80
WID = 16
NPASS = D_IN // WID
E_PAD = NW * CH * CW
ROWS = 40960
RPW = ROWS // NS
LBL_PW = 3136
E_LBL_PAD = NW * LBL_PW

_MESH = dict(core_axis_name="c", subcore_axis_name="s",
             num_cores=NC, num_subcores=NS)


def _fill_zeros(ref, nrows, width):
    z16 = jnp.zeros((16,), jnp.float32)

    def body(i, _):
        for m in range(width // 16):
            ref[i, pl.ds(m * 16, 16)] = z16
        return 0

    lax.fori_loop(0, nrows, body, 0, unroll=False)


def _sc_counts_body(dst3, typ3, kidx_out, cnt_out,
                    kidxv, rowbuf, onesb, zbufb, acc_cnt):
    cid = lax.axis_index("c")
    sid = lax.axis_index("s")
    wid = cid * NS + sid
    r0 = sid * RPW

    def p1(j, _):
        pltpu.sync_copy(dst3.at[wid, j], rowbuf.at[0])
        pltpu.sync_copy(typ3.at[wid, j], rowbuf.at[1])
        for m in range(CW // 16):
            d = rowbuf[0, pl.ds(m * 16, 16)]
            t = rowbuf[1, pl.ds(m * 16, 16)]
            kidxv[j, pl.ds(m * 16, 16)] = t * N + d
        return 0

    lax.fori_loop(0, CH, p1, 0, unroll=False)
    pltpu.sync_copy(kidxv, kidx_out.at[wid])

    def fill_ones(i, _):
        onesb[i] = jnp.ones((16,), jnp.float32)
        return 0

    lax.fori_loop(0, CW, fill_ones, 0, unroll=False)
    _fill_zeros(zbufb, 512, 16)
    for q in range(RPW // 512):
        pltpu.sync_copy(zbufb, acc_cnt.at[pl.ds(r0 + q * 512, 512)])
    plsc.subcore_barrier()

    def p2(j, _):
        pltpu.sync_copy(onesb, acc_cnt.at[kidxv.at[j]], add=True)
        return 0

    lax.fori_loop(0, CH, p2, 0, unroll=False)
    plsc.subcore_barrier()
    pltpu.sync_copy(acc_cnt.at[pl.ds(r0, RPW)],
                    cnt_out.at[cid, pl.ds(r0, RPW)])


def _sc_counts(dst3, typ3):
    return pl.kernel(
        _sc_counts_body,
        out_type=[
            jax.ShapeDtypeStruct((NW, CH, CW), jnp.int32),
            jax.ShapeDtypeStruct((NC, ROWS, 16), jnp.float32),
        ],
        mesh=plsc.VectorSubcoreMesh(**_MESH),
        compiler_params=pltpu.CompilerParams(use_tc_tiling_on_sc=False),
        scratch_types=[
            pltpu.VMEM((CH, CW), jnp.int32),
            pltpu.VMEM((2, CW), jnp.int32),
            pltpu.VMEM((CW, 16), jnp.float32),
            pltpu.VMEM((512, 16), jnp.float32),
            pltpu.VMEM_SHARED((ROWS, 16), jnp.float32),
        ],
    )(dst3, typ3)


def _sc_layer_body(*args):
    tables = list(args[:NPASS])
    (src3, kidx3, s_out, srcv, kidxv, zbufa, gbuf, gsem, ssem, acc) = args[NPASS:]
    cid = lax.axis_index("c")
    sid = lax.axis_index("s")
    wid = cid * NS + sid
    r0 = sid * RPW

    pltpu.sync_copy(src3.at[wid], srcv)
    pltpu.sync_copy(kidx3.at[wid], kidxv)
    _fill_zeros(zbufa, RPW // 2, WID)

    NB, G = 8, 4
    for c, tc in enumerate(tables):
        pltpu.sync_copy(zbufa, acc.at[pl.ds(r0, RPW // 2)])
        pltpu.sync_copy(zbufa, acc.at[pl.ds(r0 + RPW // 2, RPW // 2)])
        plsc.subcore_barrier()

        for b in range(G):
            pltpu.async_copy(tc.at[srcv.at[b]], gbuf.at[b], gsem)

        def p3(i8, _):
            for b in range(NB):
                j = i8 * NB + b
                jn = j + G

                @pl.when(j >= G)
                def _():
                    pltpu.make_async_copy(
                        gbuf.at[(b + G) % NB],
                        acc.at[kidxv.at[j - G]], ssem).wait()

                @pl.when(jn < CH)
                def _():
                    pltpu.async_copy(tc.at[srcv.at[jn]],
                                     gbuf.at[(b + G) % NB], gsem)

                pltpu.make_async_copy(tc.at[srcv.at[j]],
                                      gbuf.at[b], gsem).wait()
                pltpu.async_copy(gbuf.at[b], acc.at[kidxv.at[j]], ssem,
                                 add=True)
            return 0

        lax.fori_loop(0, CH // NB, p3, 0, unroll=False)
        for t in range(G):
            pltpu.make_async_copy(gbuf.at[t], acc.at[kidxv.at[CH - G + t]],
                                  ssem).wait()
        plsc.subcore_barrier()
        pltpu.sync_copy(acc.at[pl.ds(r0, RPW)],
                        s_out.at[cid, pl.ds(r0, RPW), pl.ds(c * WID, WID)])
        plsc.subcore_barrier()


def _sc_layer(tables, src3, kidx3):
    return pl.kernel(
        _sc_layer_body,
        out_type=[
            jax.ShapeDtypeStruct((NC, ROWS, D_IN), jnp.float32),
        ],
        mesh=plsc.VectorSubcoreMesh(**_MESH),
        compiler_params=pltpu.CompilerParams(use_tc_tiling_on_sc=False),
        scratch_types=[
            pltpu.VMEM((CH, CW), jnp.int32),
            pltpu.VMEM((CH, CW), jnp.int32),
            pltpu.VMEM((RPW // 2, WID), jnp.float32),
            pltpu.VMEM((8, CW, WID), jnp.float32),
            pltpu.SemaphoreType.DMA,
            pltpu.SemaphoreType.DMA,
            pltpu.VMEM_SHARED((ROWS, WID), jnp.float32),
        ],
    )(*tables, src3, kidx3)[0]


def _sc_decode_body(scores, ls, ld, out, sc_v, lsv, ldv, ob):
    cid = lax.axis_index("c")
    sid = lax.axis_index("s")
    wid = cid * NS + sid
    pltpu.sync_copy(scores, sc_v)
    pltpu.sync_copy(ls.at[wid], lsv)
    pltpu.sync_copy(ld.at[wid], ldv)

    def body(i, _):
        s = lsv[pl.ds(i * 16, 16)]
        d = ldv[pl.ds(i * 16, 16)]
        a = plsc.load_gather(sc_v, [s * 2])
        b = plsc.load_gather(sc_v, [d * 2 + 1])
        ob[pl.ds(i * 16, 16)] = a + b
        return 0

    lax.fori_loop(0, LBL_PW // 16, body, 0, unroll=False)
    pltpu.sync_copy(ob, out.at[wid])


def _sc_decode(scores, ls, ld):
    return pl.kernel(
        _sc_decode_body,
        out_type=[jax.ShapeDtypeStruct((NW, LBL_PW), jnp.float32)],
        mesh=plsc.VectorSubcoreMesh(**_MESH),
        compiler_params=pltpu.CompilerParams(use_tc_tiling_on_sc=False,
                                             needs_layout_passes=False),
        scratch_types=[
            pltpu.VMEM((2 * N,), jnp.float32),
            pltpu.VMEM((LBL_PW,), jnp.int32),
            pltpu.VMEM((LBL_PW,), jnp.int32),
            pltpu.VMEM((LBL_PW,), jnp.float32),
        ],
    )(scores, ls, ld)[0]


_BLK = 1000
_NBLK = N // _BLK


def _proj_body(x0_ref, x1_ref, w0_ref, b0_ref, w1_ref, b1_ref,
               xc_ref, *chunk_refs):
    i = pl.program_id(0)
    y0 = jnp.dot(x0_ref[...], w0_ref[...],
                 preferred_element_type=jnp.float32) + b0_ref[...]
    y1 = jnp.dot(x1_ref[...], w1_ref[...],
                 preferred_element_type=jnp.float32) + b1_ref[...]
    y = jnp.where(i < _NBLK // 2, y0, y1)
    xc_ref[...] = y
    for c, ref in enumerate(chunk_refs):
        ref[...] = y[:, c * WID:(c + 1) * WID]


def _proj(x0, x1, w0, b0, w1, b1):
    half = _NBLK // 2
    chunk_spec = pl.BlockSpec((_BLK, WID), lambda i: (i, 0))
    return pl.pallas_call(
        _proj_body,
        grid=(_NBLK,),
        in_specs=[
            pl.BlockSpec((_BLK, D_IN), lambda i: (jnp.minimum(i, half - 1), 0)),
            pl.BlockSpec((_BLK, D_IN), lambda i: (jnp.maximum(i - half, 0), 0)),
            pl.BlockSpec((D_IN, D_IN), lambda i: (0, 0)),
            pl.BlockSpec((1, D_IN), lambda i: (0, 0)),
            pl.BlockSpec((D_IN, D_IN), lambda i: (0, 0)),
            pl.BlockSpec((1, D_IN), lambda i: (0, 0)),
        ],
        out_specs=[
            pl.BlockSpec((_BLK, D_IN), lambda i: (i, 0)),
        ] + [chunk_spec] * NPASS,
        out_shape=[
            jax.ShapeDtypeStruct((N, D_IN), jnp.float32),
        ] + [jax.ShapeDtypeStruct((N, WID), jnp.float32)] * NPASS,
    )(x0, x1, w0, b0, w1, b1)


def _wbuild_body(c1_ref, b1_ref, c2_ref, b2_ref, w1_ref, w2_ref):
    w1_ref[...] = jnp.dot(c1_ref[...], b1_ref[...],
                          preferred_element_type=jnp.float32)
    w2_ref[...] = jnp.dot(c2_ref[...], b2_ref[...],
                          preferred_element_type=jnp.float32)


def _wbuild(comp1, b1f, comp2, b2f):
    return pl.pallas_call(
        _wbuild_body,
        out_shape=[
            jax.ShapeDtypeStruct((NREL, D_IN * D_HID), jnp.float32),
            jax.ShapeDtypeStruct((NREL, D_HID * D_OUT), jnp.float32),
        ],
    )(comp1, b1f, comp2, b2f)


def _layer_tc_body(dout, last, x_ref, s0, s1, s2, s3, c0, c1, c2, c3,
                   w_ref, root_ref, bias_ref, *out_refs):
    acc = jnp.dot(x_ref[...], root_ref[...],
                  preferred_element_type=jnp.float32) + bias_ref[...]
    for r, (s_ref, c_ref) in enumerate(zip((s0, s1, s2, s3),
                                           (c0, c1, c2, c3))):
        h = s_ref[0] + s_ref[1]
        cnt = c_ref[0, :, 0] + c_ref[1, :, 0]
        inv = 1.0 / jnp.maximum(cnt, 1.0)
        h = h * inv[:, None]
        acc = acc + jnp.dot(h, w_ref[r], preferred_element_type=jnp.float32)
    if not last:
        y = jnp.maximum(acc, 0.0)
        out_refs[0][...] = y
        for c in range(NPASS):
            out_refs[1 + c][...] = y[:, c * WID:(c + 1) * WID]
    else:
        fct_ref, fcb_ref, scores_ref = out_refs
        scores = jnp.dot(acc, fct_ref[...],
                         preferred_element_type=jnp.float32) + fcb_ref[...]
        scores_ref[...] = scores


def _layer_tc(x, s, cnt, w, root, bias, last, fct=None, fcb=None):
    dout = D_OUT if last else D_HID

    def s_spec(r):
        return pl.BlockSpec((NC, _BLK, D_IN),
                            lambda i, r=r: (0, r * _NBLK + i, 0))

    def c_spec(r):
        return pl.BlockSpec((NC, _BLK, 16),
                            lambda i, r=r: (0, r * _NBLK + i, 0))

    in_specs = [pl.BlockSpec((_BLK, x.shape[1]), lambda i: (i, 0))]
    in_specs += [s_spec(r) for r in range(NREL)]
    in_specs += [c_spec(r) for r in range(NREL)]
    in_specs += [
        pl.BlockSpec((NREL, w.shape[1], dout), lambda i: (0, 0, 0)),
        pl.BlockSpec((root.shape[0], dout), lambda i: (0, 0)),
        pl.BlockSpec((1, dout), lambda i: (0, 0)),
    ]
    args = [x, s, s, s, s, cnt, cnt, cnt, cnt, w, root, bias]
    if not last:
        out_specs = [pl.BlockSpec((_BLK, dout), lambda i: (i, 0))]
        out_specs += [pl.BlockSpec((_BLK, WID), lambda i: (i, 0))] * NPASS
        out_shape = [jax.ShapeDtypeStruct((N, dout), jnp.float32)]
        out_shape += [jax.ShapeDtypeStruct((N, WID), jnp.float32)] * NPASS
    else:
        in_specs += [
            pl.BlockSpec((D_OUT, 2), lambda i: (0, 0)),
            pl.BlockSpec((1, 2), lambda i: (0, 0)),
        ]
        args += [fct, fcb]
        out_specs = [pl.BlockSpec((_BLK, 2), lambda i: (i, 0))]
        out_shape = [jax.ShapeDtypeStruct((N, 2), jnp.float32)]
    return pl.pallas_call(
        functools.partial(_layer_tc_body, dout, last),
        grid=(_NBLK,),
        in_specs=in_specs,
        out_specs=out_specs,
        out_shape=out_shape,
    )(*args)


def kernel(x0, x1, edge_index, edge_type, edge_label_index,
           lin0_w, lin0_b, lin1_w, lin1_b,
           bases1, comp1, root1, bias1,
           bases2, comp2, root2, bias2,
           fc_w, fc_b):
    f32 = jnp.float32
    src = edge_index[0]
    dst = edge_index[1]
    pad = E_PAD - E
    src3 = jnp.pad(src, (0, pad)).reshape(NW, CH, CW)
    dst3 = jnp.pad(dst, (0, pad)).reshape(NW, CH, CW)
    typ3 = jnp.pad(edge_type, (0, pad), constant_values=NREL).reshape(NW, CH, CW)
    ls = jnp.pad(edge_label_index[0], (0, E_LBL_PAD - E_LBL)).reshape(NW, LBL_PW)
    ld = jnp.pad(edge_label_index[1], (0, E_LBL_PAD - E_LBL)).reshape(NW, LBL_PW)

    xcat, *xchunks = _proj(
        x0, x1, lin0_w, lin0_b.reshape(1, -1), lin1_w, lin1_b.reshape(1, -1))
    w1f, w2f = _wbuild(comp1, bases1.reshape(30, -1), comp2,
                       bases2.reshape(30, -1))
    w1 = w1f.reshape(NREL, D_IN, D_HID)
    w2 = w2f.reshape(NREL, D_HID, D_OUT)

    kidx3, cnt = _sc_counts(dst3, typ3)
    s1 = _sc_layer(xchunks, src3, kidx3)
    z, *zchunks = _layer_tc(
        xcat, s1, cnt, w1, root1, bias1.reshape(1, -1), last=False)
    s2 = _sc_layer(zchunks, src3, kidx3)

    fct = jnp.stack([fc_w[:D_OUT, 0], fc_w[D_OUT:, 0]], axis=1)
    fcb = jnp.concatenate([fc_b, jnp.zeros((1,), f32)]).reshape(1, 2)
    (scores,) = _layer_tc(z, s2, cnt, w2, root2, bias2.reshape(1, -1),
                          last=True, fct=fct, fcb=fcb)

    out = _sc_decode(scores.reshape(-1), ls, ld)
    return out.reshape(-1)[:E_LBL].reshape(E_LBL, 1)


# speedup vs baseline: 6.9016x; 1.0017x over previous
import functools

import jax
import jax.numpy as jnp
from jax import lax
from jax.experimental import pallas as pl
from jax.experimental.pallas import tpu as pltpu
from jax.experimental.pallas import tpu_sc as plsc

N = 10000
D_IN = 128
D_HID = 128
D_OUT = 64
NREL = 4
E = 320000
E_LBL = 100000

NC, NS, L = 2, 16, 16
NW = NC * NS
CW = 128
CH = 80
WID = 16
NPASS = D_IN // WID
E_PAD = NW * CH * CW
ROWS = 40960
RPW = ROWS // NS
LBL_PW = 3136
E_LBL_PAD = NW * LBL_PW

_MESH = dict(core_axis_name="c", subcore_axis_name="s",
             num_cores=NC, num_subcores=NS)


def _fill_zeros(ref, nrows, width):
    z16 = jnp.zeros((16,), jnp.float32)

    def body(i, _):
        for m in range(width // 16):
            ref[i, pl.ds(m * 16, 16)] = z16
        return 0

    lax.fori_loop(0, nrows, body, 0, unroll=False)


def _sc_counts_body(dst3, typ3, kidx_out, cnt_out,
                    kidxv, dstv, typv, onesb, zbufb, sem, acc_cnt):
    cid = lax.axis_index("c")
    sid = lax.axis_index("s")
    wid = cid * NS + sid
    r0 = sid * RPW

    pltpu.sync_copy(dst3.at[wid], dstv)
    pltpu.sync_copy(typ3.at[wid], typv)

    def p1(j, _):
        for m in range(CW // 16):
            d = dstv[j, pl.ds(m * 16, 16)]
            t = typv[j, pl.ds(m * 16, 16)]
            kidxv[j, pl.ds(m * 16, 16)] = t * N + d
        return 0

    lax.fori_loop(0, CH, p1, 0, unroll=False)
    pltpu.sync_copy(kidxv, kidx_out.at[wid])

    def fill_ones(i, _):
        onesb[i] = jnp.ones((16,), jnp.float32)
        return 0

    lax.fori_loop(0, CW, fill_ones, 0, unroll=False)
    _fill_zeros(zbufb, 512, 16)
    for q in range(RPW // 512):
        pltpu.sync_copy(zbufb, acc_cnt.at[pl.ds(r0 + q * 512, 512)])
    plsc.subcore_barrier()

    def p2(j, _):
        pltpu.async_copy(onesb, acc_cnt.at[kidxv.at[j]], sem, add=True)
        return 0

    lax.fori_loop(0, CH, p2, 0, unroll=False)

    def p2d(j, _):
        pltpu.make_async_copy(onesb, acc_cnt.at[kidxv.at[j]], sem).wait()
        return 0

    lax.fori_loop(0, CH, p2d, 0, unroll=False)
    plsc.subcore_barrier()
    pltpu.sync_copy(acc_cnt.at[pl.ds(r0, RPW)],
                    cnt_out.at[cid, pl.ds(r0, RPW)])


def _sc_counts(dst3, typ3):
    return pl.kernel(
        _sc_counts_body,
        out_type=[
            jax.ShapeDtypeStruct((NW, CH, CW), jnp.int32),
            jax.ShapeDtypeStruct((NC, ROWS, 16), jnp.float32),
        ],
        mesh=plsc.VectorSubcoreMesh(**_MESH),
        compiler_params=pltpu.CompilerParams(use_tc_tiling_on_sc=False),
        scratch_types=[
            pltpu.VMEM((CH, CW), jnp.int32),
            pltpu.VMEM((CH, CW), jnp.int32),
            pltpu.VMEM((CH, CW), jnp.int32),
            pltpu.VMEM((CW, 16), jnp.float32),
            pltpu.VMEM((512, 16), jnp.float32),
            pltpu.SemaphoreType.DMA,
            pltpu.VMEM_SHARED((ROWS, 16), jnp.float32),
        ],
    )(dst3, typ3)


def _sc_layer_body(*args):
    tables = list(args[:NPASS])
    (src3, kidx3, s_out, srcv, kidxv, zbufa, gbuf, gsem, ssem, acc) = args[NPASS:]
    cid = lax.axis_index("c")
    sid = lax.axis_index("s")
    wid = cid * NS + sid
    r0 = sid * RPW

    pltpu.sync_copy(src3.at[wid], srcv)
    pltpu.sync_copy(kidx3.at[wid], kidxv)
    _fill_zeros(zbufa, RPW // 2, WID)

    NB, G = 8, 4
    for c, tc in enumerate(tables):
        pltpu.sync_copy(zbufa, acc.at[pl.ds(r0, RPW // 2)])
        pltpu.sync_copy(zbufa, acc.at[pl.ds(r0 + RPW // 2, RPW // 2)])
        plsc.subcore_barrier()

        for b in range(G):
            pltpu.async_copy(tc.at[srcv.at[b]], gbuf.at[b], gsem)

        def p3(i8, _):
            for b in range(NB):
                j = i8 * NB + b
                jn = j + G

                @pl.when(j >= G)
                def _():
                    pltpu.make_async_copy(
                        gbuf.at[(b + G) % NB],
                        acc.at[kidxv.at[j - G]], ssem).wait()

                @pl.when(jn < CH)
                def _():
                    pltpu.async_copy(tc.at[srcv.at[jn]],
                                     gbuf.at[(b + G) % NB], gsem)

                pltpu.make_async_copy(tc.at[srcv.at[j]],
                                      gbuf.at[b], gsem).wait()
                pltpu.async_copy(gbuf.at[b], acc.at[kidxv.at[j]], ssem,
                                 add=True)
            return 0

        lax.fori_loop(0, CH // NB, p3, 0, unroll=False)
        for t in range(G):
            pltpu.make_async_copy(gbuf.at[t], acc.at[kidxv.at[CH - G + t]],
                                  ssem).wait()
        plsc.subcore_barrier()
        pltpu.sync_copy(acc.at[pl.ds(r0, RPW)],
                        s_out.at[cid, pl.ds(r0, RPW), pl.ds(c * WID, WID)])
        plsc.subcore_barrier()


def _sc_layer(tables, src3, kidx3):
    return pl.kernel(
        _sc_layer_body,
        out_type=[
            jax.ShapeDtypeStruct((NC, ROWS, D_IN), jnp.float32),
        ],
        mesh=plsc.VectorSubcoreMesh(**_MESH),
        compiler_params=pltpu.CompilerParams(use_tc_tiling_on_sc=False),
        scratch_types=[
            pltpu.VMEM((CH, CW), jnp.int32),
            pltpu.VMEM((CH, CW), jnp.int32),
            pltpu.VMEM((RPW // 2, WID), jnp.float32),
            pltpu.VMEM((8, CW, WID), jnp.float32),
            pltpu.SemaphoreType.DMA,
            pltpu.SemaphoreType.DMA,
            pltpu.VMEM_SHARED((ROWS, WID), jnp.float32),
        ],
    )(*tables, src3, kidx3)[0]


def _sc_decode_body(scores, ls, ld, out, sc_v, lsv, ldv, ob):
    cid = lax.axis_index("c")
    sid = lax.axis_index("s")
    wid = cid * NS + sid
    pltpu.sync_copy(scores, sc_v)
    pltpu.sync_copy(ls.at[wid], lsv)
    pltpu.sync_copy(ld.at[wid], ldv)

    def body(i, _):
        s = lsv[pl.ds(i * 16, 16)]
        d = ldv[pl.ds(i * 16, 16)]
        a = plsc.load_gather(sc_v, [s * 2])
        b = plsc.load_gather(sc_v, [d * 2 + 1])
        ob[pl.ds(i * 16, 16)] = a + b
        return 0

    lax.fori_loop(0, LBL_PW // 16, body, 0, unroll=False)
    pltpu.sync_copy(ob, out.at[wid])


def _sc_decode(scores, ls, ld):
    return pl.kernel(
        _sc_decode_body,
        out_type=[jax.ShapeDtypeStruct((NW, LBL_PW), jnp.float32)],
        mesh=plsc.VectorSubcoreMesh(**_MESH),
        compiler_params=pltpu.CompilerParams(use_tc_tiling_on_sc=False,
                                             needs_layout_passes=False),
        scratch_types=[
            pltpu.VMEM((2 * N,), jnp.float32),
            pltpu.VMEM((LBL_PW,), jnp.int32),
            pltpu.VMEM((LBL_PW,), jnp.int32),
            pltpu.VMEM((LBL_PW,), jnp.float32),
        ],
    )(scores, ls, ld)[0]


_BLK = 1000
_NBLK = N // _BLK


def _proj_body(x0_ref, x1_ref, w0_ref, b0_ref, w1_ref, b1_ref,
               xc_ref, *chunk_refs):
    i = pl.program_id(0)
    y0 = jnp.dot(x0_ref[...], w0_ref[...],
                 preferred_element_type=jnp.float32) + b0_ref[...]
    y1 = jnp.dot(x1_ref[...], w1_ref[...],
                 preferred_element_type=jnp.float32) + b1_ref[...]
    y = jnp.where(i < _NBLK // 2, y0, y1)
    xc_ref[...] = y
    for c, ref in enumerate(chunk_refs):
        ref[...] = y[:, c * WID:(c + 1) * WID]


def _proj(x0, x1, w0, b0, w1, b1):
    half = _NBLK // 2
    chunk_spec = pl.BlockSpec((_BLK, WID), lambda i: (i, 0))
    return pl.pallas_call(
        _proj_body,
        grid=(_NBLK,),
        in_specs=[
            pl.BlockSpec((_BLK, D_IN), lambda i: (jnp.minimum(i, half - 1), 0)),
            pl.BlockSpec((_BLK, D_IN), lambda i: (jnp.maximum(i - half, 0), 0)),
            pl.BlockSpec((D_IN, D_IN), lambda i: (0, 0)),
            pl.BlockSpec((1, D_IN), lambda i: (0, 0)),
            pl.BlockSpec((D_IN, D_IN), lambda i: (0, 0)),
            pl.BlockSpec((1, D_IN), lambda i: (0, 0)),
        ],
        out_specs=[
            pl.BlockSpec((_BLK, D_IN), lambda i: (i, 0)),
        ] + [chunk_spec] * NPASS,
        out_shape=[
            jax.ShapeDtypeStruct((N, D_IN), jnp.float32),
        ] + [jax.ShapeDtypeStruct((N, WID), jnp.float32)] * NPASS,
    )(x0, x1, w0, b0, w1, b1)


def _wbuild_body(c1_ref, b1_ref, c2_ref, b2_ref, w1_ref, w2_ref):
    w1_ref[...] = jnp.dot(c1_ref[...], b1_ref[...],
                          preferred_element_type=jnp.float32)
    w2_ref[...] = jnp.dot(c2_ref[...], b2_ref[...],
                          preferred_element_type=jnp.float32)


def _wbuild(comp1, b1f, comp2, b2f):
    return pl.pallas_call(
        _wbuild_body,
        out_shape=[
            jax.ShapeDtypeStruct((NREL, D_IN * D_HID), jnp.float32),
            jax.ShapeDtypeStruct((NREL, D_HID * D_OUT), jnp.float32),
        ],
    )(comp1, b1f, comp2, b2f)


def _layer_tc_body(dout, last, x_ref, s0, s1, s2, s3, c0, c1, c2, c3,
                   w_ref, root_ref, bias_ref, *out_refs):
    acc = jnp.dot(x_ref[...], root_ref[...],
                  preferred_element_type=jnp.float32) + bias_ref[...]
    for r, (s_ref, c_ref) in enumerate(zip((s0, s1, s2, s3),
                                           (c0, c1, c2, c3))):
        h = s_ref[0] + s_ref[1]
        cnt = c_ref[0, :, 0] + c_ref[1, :, 0]
        inv = 1.0 / jnp.maximum(cnt, 1.0)
        h = h * inv[:, None]
        acc = acc + jnp.dot(h, w_ref[r], preferred_element_type=jnp.float32)
    if not last:
        y = jnp.maximum(acc, 0.0)
        out_refs[0][...] = y
        for c in range(NPASS):
            out_refs[1 + c][...] = y[:, c * WID:(c + 1) * WID]
    else:
        fct_ref, fcb_ref, scores_ref = out_refs
        scores = jnp.dot(acc, fct_ref[...],
                         preferred_element_type=jnp.float32) + fcb_ref[...]
        scores_ref[...] = scores


def _layer_tc(x, s, cnt, w, root, bias, last, fct=None, fcb=None):
    dout = D_OUT if last else D_HID

    def s_spec(r):
        return pl.BlockSpec((NC, _BLK, D_IN),
                            lambda i, r=r: (0, r * _NBLK + i, 0))

    def c_spec(r):
        return pl.BlockSpec((NC, _BLK, 16),
                            lambda i, r=r: (0, r * _NBLK + i, 0))

    in_specs = [pl.BlockSpec((_BLK, x.shape[1]), lambda i: (i, 0))]
    in_specs += [s_spec(r) for r in range(NREL)]
    in_specs += [c_spec(r) for r in range(NREL)]
    in_specs += [
        pl.BlockSpec((NREL, w.shape[1], dout), lambda i: (0, 0, 0)),
        pl.BlockSpec((root.shape[0], dout), lambda i: (0, 0)),
        pl.BlockSpec((1, dout), lambda i: (0, 0)),
    ]
    args = [x, s, s, s, s, cnt, cnt, cnt, cnt, w, root, bias]
    if not last:
        out_specs = [pl.BlockSpec((_BLK, dout), lambda i: (i, 0))]
        out_specs += [pl.BlockSpec((_BLK, WID), lambda i: (i, 0))] * NPASS
        out_shape = [jax.ShapeDtypeStruct((N, dout), jnp.float32)]
        out_shape += [jax.ShapeDtypeStruct((N, WID), jnp.float32)] * NPASS
    else:
        in_specs += [
            pl.BlockSpec((D_OUT, 2), lambda i: (0, 0)),
            pl.BlockSpec((1, 2), lambda i: (0, 0)),
        ]
        args += [fct, fcb]
        out_specs = [pl.BlockSpec((_BLK, 2), lambda i: (i, 0))]
        out_shape = [jax.ShapeDtypeStruct((N, 2), jnp.float32)]
    return pl.pallas_call(
        functools.partial(_layer_tc_body, dout, last),
        grid=(_NBLK,),
        in_specs=in_specs,
        out_specs=out_specs,
        out_shape=out_shape,
    )(*args)


def kernel(x0, x1, edge_index, edge_type, edge_label_index,
           lin0_w, lin0_b, lin1_w, lin1_b,
           bases1, comp1, root1, bias1,
           bases2, comp2, root2, bias2,
           fc_w, fc_b):
    f32 = jnp.float32
    src = edge_index[0]
    dst = edge_index[1]
    pad = E_PAD - E
    src3 = jnp.pad(src, (0, pad)).reshape(NW, CH, CW)
    dst3 = jnp.pad(dst, (0, pad)).reshape(NW, CH, CW)
    typ3 = jnp.pad(edge_type, (0, pad), constant_values=NREL).reshape(NW, CH, CW)
    trash = (jnp.arange(pad, dtype=jnp.int32) % (ROWS - NREL * N))
    dst3 = dst3.reshape(-1).at[E:].set(trash).reshape(NW, CH, CW)
    ls = jnp.pad(edge_label_index[0], (0, E_LBL_PAD - E_LBL)).reshape(NW, LBL_PW)
    ld = jnp.pad(edge_label_index[1], (0, E_LBL_PAD - E_LBL)).reshape(NW, LBL_PW)

    xcat, *xchunks = _proj(
        x0, x1, lin0_w, lin0_b.reshape(1, -1), lin1_w, lin1_b.reshape(1, -1))
    w1f, w2f = _wbuild(comp1, bases1.reshape(30, -1), comp2,
                       bases2.reshape(30, -1))
    w1 = w1f.reshape(NREL, D_IN, D_HID)
    w2 = w2f.reshape(NREL, D_HID, D_OUT)

    kidx3, cnt = _sc_counts(dst3, typ3)
    s1 = _sc_layer(xchunks, src3, kidx3)
    z, *zchunks = _layer_tc(
        xcat, s1, cnt, w1, root1, bias1.reshape(1, -1), last=False)
    s2 = _sc_layer(zchunks, src3, kidx3)

    fct = jnp.stack([fc_w[:D_OUT, 0], fc_w[D_OUT:, 0]], axis=1)
    fcb = jnp.concatenate([fc_b, jnp.zeros((1,), f32)]).reshape(1, 2)
    (scores,) = _layer_tc(z, s2, cnt, w2, root2, bias2.reshape(1, -1),
                          last=True, fct=fct, fcb=fcb)

    out = _sc_decode(scores.reshape(-1), ls, ld)
    return out.reshape(-1)[:E_LBL].reshape(E_LBL, 1)


# speedup vs baseline: 9.7912x; 1.4187x over previous
import functools

import jax
import jax.numpy as jnp
from jax import lax
from jax.experimental import pallas as pl
from jax.experimental.pallas import tpu as pltpu
from jax.experimental.pallas import tpu_sc as plsc

N = 10000
D_IN = 128
D_HID = 128
D_OUT = 64
NREL = 4
E = 320000
E_LBL = 100000

NC, NS, L = 2, 16, 16
NW = NC * NS
CW = 128
CH = 80
WID = 16
NPASS = D_IN // WID
E_PAD = NW * CH * CW
ROWS = 40960
RPW = ROWS // NS
LBL_PW = 3136
E_LBL_PAD = NW * LBL_PW

_MESH = dict(core_axis_name="c", subcore_axis_name="s",
             num_cores=NC, num_subcores=NS)


def _fill_zeros(ref, nrows, width):
    z16 = jnp.zeros((16,), jnp.float32)

    def body(i, _):
        for m in range(width // 16):
            ref[i, pl.ds(m * 16, 16)] = z16
        return 0

    lax.fori_loop(0, nrows, body, 0, unroll=False)


def _sc_counts_body(dst3, typ3, kidx_out, cnt_out,
                    kidxv, dstv, typv, onesb, zbufb, sem, acc_cnt):
    cid = lax.axis_index("c")
    sid = lax.axis_index("s")
    wid = cid * NS + sid
    r0 = sid * RPW

    pltpu.sync_copy(dst3.at[wid], dstv)
    pltpu.sync_copy(typ3.at[wid], typv)

    def p1(j, _):
        for m in range(CW // 16):
            d = dstv[j, pl.ds(m * 16, 16)]
            t = typv[j, pl.ds(m * 16, 16)]
            kidxv[j, pl.ds(m * 16, 16)] = t * N + d
        return 0

    lax.fori_loop(0, CH, p1, 0, unroll=False)
    pltpu.sync_copy(kidxv, kidx_out.at[wid])

    def fill_ones(i, _):
        onesb[i] = jnp.ones((16,), jnp.float32)
        return 0

    lax.fori_loop(0, CW, fill_ones, 0, unroll=False)
    _fill_zeros(zbufb, 512, 16)
    for q in range(RPW // 512):
        pltpu.sync_copy(zbufb, acc_cnt.at[pl.ds(r0 + q * 512, 512)])
    plsc.subcore_barrier()

    def p2(j, _):
        pltpu.async_copy(onesb, acc_cnt.at[kidxv.at[j]], sem, add=True)
        return 0

    lax.fori_loop(0, CH, p2, 0, unroll=False)

    def p2d(j, _):
        pltpu.make_async_copy(onesb, acc_cnt.at[kidxv.at[j]], sem).wait()
        return 0

    lax.fori_loop(0, CH, p2d, 0, unroll=False)
    plsc.subcore_barrier()
    pltpu.sync_copy(acc_cnt.at[pl.ds(r0, RPW)],
                    cnt_out.at[cid, pl.ds(r0, RPW)])


def _sc_counts(dst3, typ3):
    return pl.kernel(
        _sc_counts_body,
        out_type=[
            jax.ShapeDtypeStruct((NW, CH, CW), jnp.int32),
            jax.ShapeDtypeStruct((NC, ROWS, 16), jnp.float32),
        ],
        mesh=plsc.VectorSubcoreMesh(**_MESH),
        compiler_params=pltpu.CompilerParams(use_tc_tiling_on_sc=False),
        scratch_types=[
            pltpu.VMEM((CH, CW), jnp.int32),
            pltpu.VMEM((CH, CW), jnp.int32),
            pltpu.VMEM((CH, CW), jnp.int32),
            pltpu.VMEM((CW, 16), jnp.float32),
            pltpu.VMEM((512, 16), jnp.float32),
            pltpu.SemaphoreType.DMA,
            pltpu.VMEM_SHARED((ROWS, 16), jnp.float32),
        ],
    )(dst3, typ3)


CH2 = E_PAD // NS // CW


def _sc_layer_body(*args):
    tables = list(args[:NPASS])
    (src2, kidx2, s_out, srcv, kidxv, zbufa, gbuf, gsem, ssem, acc) = args[NPASS:]
    cid = lax.axis_index("c")
    sid = lax.axis_index("s")
    r0 = sid * RPW

    pltpu.sync_copy(src2.at[sid], srcv)
    pltpu.sync_copy(kidx2.at[sid], kidxv)
    _fill_zeros(zbufa, RPW // 2, WID)

    NB, G = 8, 4
    for g, tc in enumerate(tables):

        @pl.when(cid == g // (NPASS // NC))
        def _pass():
            pltpu.sync_copy(zbufa, acc.at[pl.ds(r0, RPW // 2)])
            pltpu.sync_copy(zbufa, acc.at[pl.ds(r0 + RPW // 2, RPW // 2)])
            plsc.subcore_barrier()

            for b in range(G):
                pltpu.async_copy(tc.at[srcv.at[b]], gbuf.at[b], gsem)

            def p3(i8, _):
                for b in range(NB):
                    j = i8 * NB + b
                    jn = j + G

                    @pl.when(j >= G)
                    def _():
                        pltpu.make_async_copy(
                            gbuf.at[(b + G) % NB],
                            acc.at[kidxv.at[j - G]], ssem).wait()

                    @pl.when(jn < CH2)
                    def _():
                        pltpu.async_copy(tc.at[srcv.at[jn]],
                                         gbuf.at[(b + G) % NB], gsem)

                    pltpu.make_async_copy(tc.at[srcv.at[j]],
                                          gbuf.at[b], gsem).wait()
                    pltpu.async_copy(gbuf.at[b], acc.at[kidxv.at[j]], ssem,
                                     add=True)
                return 0

            lax.fori_loop(0, CH2 // NB, p3, 0, unroll=False)
            for t in range(G):
                pltpu.make_async_copy(gbuf.at[t],
                                      acc.at[kidxv.at[CH2 - G + t]],
                                      ssem).wait()
            plsc.subcore_barrier()
            pltpu.sync_copy(acc.at[pl.ds(r0, RPW)],
                            s_out.at[pl.ds(r0, RPW), pl.ds(g * WID, WID)])
            plsc.subcore_barrier()


def _sc_layer(tables, src2, kidx2):
    return pl.kernel(
        _sc_layer_body,
        out_type=[
            jax.ShapeDtypeStruct((ROWS, D_IN), jnp.float32),
        ],
        mesh=plsc.VectorSubcoreMesh(**_MESH),
        compiler_params=pltpu.CompilerParams(use_tc_tiling_on_sc=False),
        scratch_types=[
            pltpu.VMEM((CH2, CW), jnp.int32),
            pltpu.VMEM((CH2, CW), jnp.int32),
            pltpu.VMEM((RPW // 2, WID), jnp.float32),
            pltpu.VMEM((8, CW, WID), jnp.float32),
            pltpu.SemaphoreType.DMA,
            pltpu.SemaphoreType.DMA,
            pltpu.VMEM_SHARED((ROWS, WID), jnp.float32),
        ],
    )(*tables, src2, kidx2)[0]


def _sc_decode_body(scores, ls, ld, out, sc_v, lsv, ldv, ob):
    cid = lax.axis_index("c")
    sid = lax.axis_index("s")
    wid = cid * NS + sid
    pltpu.sync_copy(scores, sc_v)
    pltpu.sync_copy(ls.at[wid], lsv)
    pltpu.sync_copy(ld.at[wid], ldv)

    def body(i, _):
        s = lsv[pl.ds(i * 16, 16)]
        d = ldv[pl.ds(i * 16, 16)]
        a = plsc.load_gather(sc_v, [s * 2])
        b = plsc.load_gather(sc_v, [d * 2 + 1])
        ob[pl.ds(i * 16, 16)] = a + b
        return 0

    lax.fori_loop(0, LBL_PW // 16, body, 0, unroll=False)
    pltpu.sync_copy(ob, out.at[wid])


def _sc_decode(scores, ls, ld):
    return pl.kernel(
        _sc_decode_body,
        out_type=[jax.ShapeDtypeStruct((NW, LBL_PW), jnp.float32)],
        mesh=plsc.VectorSubcoreMesh(**_MESH),
        compiler_params=pltpu.CompilerParams(use_tc_tiling_on_sc=False,
                                             needs_layout_passes=False),
        scratch_types=[
            pltpu.VMEM((2 * N,), jnp.float32),
            pltpu.VMEM((LBL_PW,), jnp.int32),
            pltpu.VMEM((LBL_PW,), jnp.int32),
            pltpu.VMEM((LBL_PW,), jnp.float32),
        ],
    )(scores, ls, ld)[0]


_BLK = 1000
_NBLK = N // _BLK


def _proj_body(x0_ref, x1_ref, w0_ref, b0_ref, w1_ref, b1_ref,
               xc_ref, *chunk_refs):
    i = pl.program_id(0)
    y0 = jnp.dot(x0_ref[...], w0_ref[...],
                 preferred_element_type=jnp.float32) + b0_ref[...]
    y1 = jnp.dot(x1_ref[...], w1_ref[...],
                 preferred_element_type=jnp.float32) + b1_ref[...]
    y = jnp.where(i < _NBLK // 2, y0, y1)
    xc_ref[...] = y
    for c, ref in enumerate(chunk_refs):
        ref[...] = y[:, c * WID:(c + 1) * WID]


def _proj(x0, x1, w0, b0, w1, b1):
    half = _NBLK // 2
    chunk_spec = pl.BlockSpec((_BLK, WID), lambda i: (i, 0))
    return pl.pallas_call(
        _proj_body,
        grid=(_NBLK,),
        in_specs=[
            pl.BlockSpec((_BLK, D_IN), lambda i: (jnp.minimum(i, half - 1), 0)),
            pl.BlockSpec((_BLK, D_IN), lambda i: (jnp.maximum(i - half, 0), 0)),
            pl.BlockSpec((D_IN, D_IN), lambda i: (0, 0)),
            pl.BlockSpec((1, D_IN), lambda i: (0, 0)),
            pl.BlockSpec((D_IN, D_IN), lambda i: (0, 0)),
            pl.BlockSpec((1, D_IN), lambda i: (0, 0)),
        ],
        out_specs=[
            pl.BlockSpec((_BLK, D_IN), lambda i: (i, 0)),
        ] + [chunk_spec] * NPASS,
        out_shape=[
            jax.ShapeDtypeStruct((N, D_IN), jnp.float32),
        ] + [jax.ShapeDtypeStruct((N, WID), jnp.float32)] * NPASS,
    )(x0, x1, w0, b0, w1, b1)


def _wbuild_body(c1_ref, b1_ref, c2_ref, b2_ref, w1_ref, w2_ref):
    w1_ref[...] = jnp.dot(c1_ref[...], b1_ref[...],
                          preferred_element_type=jnp.float32)
    w2_ref[...] = jnp.dot(c2_ref[...], b2_ref[...],
                          preferred_element_type=jnp.float32)


def _wbuild(comp1, b1f, comp2, b2f):
    return pl.pallas_call(
        _wbuild_body,
        out_shape=[
            jax.ShapeDtypeStruct((NREL, D_IN * D_HID), jnp.float32),
            jax.ShapeDtypeStruct((NREL, D_HID * D_OUT), jnp.float32),
        ],
    )(comp1, b1f, comp2, b2f)


def _layer_tc_body(dout, last, x_ref, s0, s1, s2, s3, c0, c1, c2, c3,
                   w_ref, root_ref, bias_ref, *out_refs):
    acc = jnp.dot(x_ref[...], root_ref[...],
                  preferred_element_type=jnp.float32) + bias_ref[...]
    for r, (s_ref, c_ref) in enumerate(zip((s0, s1, s2, s3),
                                           (c0, c1, c2, c3))):
        h = s_ref[...]
        cnt = c_ref[0, :, 0] + c_ref[1, :, 0]
        inv = 1.0 / jnp.maximum(cnt, 1.0)
        h = h * inv[:, None]
        acc = acc + jnp.dot(h, w_ref[r], preferred_element_type=jnp.float32)
    if not last:
        y = jnp.maximum(acc, 0.0)
        out_refs[0][...] = y
        for c in range(NPASS):
            out_refs[1 + c][...] = y[:, c * WID:(c + 1) * WID]
    else:
        fct_ref, fcb_ref, scores_ref = out_refs
        scores = jnp.dot(acc, fct_ref[...],
                         preferred_element_type=jnp.float32) + fcb_ref[...]
        scores_ref[...] = scores


def _layer_tc(x, s, cnt, w, root, bias, last, fct=None, fcb=None):
    dout = D_OUT if last else D_HID

    def s_spec(r):
        return pl.BlockSpec((_BLK, D_IN),
                            lambda i, r=r: (r * _NBLK + i, 0))

    def c_spec(r):
        return pl.BlockSpec((NC, _BLK, 16),
                            lambda i, r=r: (0, r * _NBLK + i, 0))

    in_specs = [pl.BlockSpec((_BLK, x.shape[1]), lambda i: (i, 0))]
    in_specs += [s_spec(r) for r in range(NREL)]
    in_specs += [c_spec(r) for r in range(NREL)]
    in_specs += [
        pl.BlockSpec((NREL, w.shape[1], dout), lambda i: (0, 0, 0)),
        pl.BlockSpec((root.shape[0], dout), lambda i: (0, 0)),
        pl.BlockSpec((1, dout), lambda i: (0, 0)),
    ]
    args = [x, s, s, s, s, cnt, cnt, cnt, cnt, w, root, bias]
    if not last:
        out_specs = [pl.BlockSpec((_BLK, dout), lambda i: (i, 0))]
        out_specs += [pl.BlockSpec((_BLK, WID), lambda i: (i, 0))] * NPASS
        out_shape = [jax.ShapeDtypeStruct((N, dout), jnp.float32)]
        out_shape += [jax.ShapeDtypeStruct((N, WID), jnp.float32)] * NPASS
    else:
        in_specs += [
            pl.BlockSpec((D_OUT, 2), lambda i: (0, 0)),
            pl.BlockSpec((1, 2), lambda i: (0, 0)),
        ]
        args += [fct, fcb]
        out_specs = [pl.BlockSpec((_BLK, 2), lambda i: (i, 0))]
        out_shape = [jax.ShapeDtypeStruct((N, 2), jnp.float32)]
    return pl.pallas_call(
        functools.partial(_layer_tc_body, dout, last),
        grid=(_NBLK,),
        in_specs=in_specs,
        out_specs=out_specs,
        out_shape=out_shape,
    )(*args)


def kernel(x0, x1, edge_index, edge_type, edge_label_index,
           lin0_w, lin0_b, lin1_w, lin1_b,
           bases1, comp1, root1, bias1,
           bases2, comp2, root2, bias2,
           fc_w, fc_b):
    f32 = jnp.float32
    src = edge_index[0]
    dst = edge_index[1]
    pad = E_PAD - E
    src3 = jnp.pad(src, (0, pad)).reshape(NW, CH, CW)
    dst3 = jnp.pad(dst, (0, pad)).reshape(NW, CH, CW)
    typ3 = jnp.pad(edge_type, (0, pad), constant_values=NREL).reshape(NW, CH, CW)
    trash = (jnp.arange(pad, dtype=jnp.int32) % (ROWS - NREL * N))
    dst3 = dst3.reshape(-1).at[E:].set(trash).reshape(NW, CH, CW)
    ls = jnp.pad(edge_label_index[0], (0, E_LBL_PAD - E_LBL)).reshape(NW, LBL_PW)
    ld = jnp.pad(edge_label_index[1], (0, E_LBL_PAD - E_LBL)).reshape(NW, LBL_PW)

    xcat, *xchunks = _proj(
        x0, x1, lin0_w, lin0_b.reshape(1, -1), lin1_w, lin1_b.reshape(1, -1))
    w1f, w2f = _wbuild(comp1, bases1.reshape(30, -1), comp2,
                       bases2.reshape(30, -1))
    w1 = w1f.reshape(NREL, D_IN, D_HID)
    w2 = w2f.reshape(NREL, D_HID, D_OUT)

    kidx3, cnt = _sc_counts(dst3, typ3)
    src2 = src3.reshape(NS, CH2, CW)
    kidx2 = kidx3.reshape(NS, CH2, CW)
    s1 = _sc_layer(xchunks, src2, kidx2)
    z, *zchunks = _layer_tc(
        xcat, s1, cnt, w1, root1, bias1.reshape(1, -1), last=False)
    s2 = _sc_layer(zchunks, src2, kidx2)

    fct = jnp.stack([fc_w[:D_OUT, 0], fc_w[D_OUT:, 0]], axis=1)
    fcb = jnp.concatenate([fc_b, jnp.zeros((1,), f32)]).reshape(1, 2)
    (scores,) = _layer_tc(z, s2, cnt, w2, root2, bias2.reshape(1, -1),
                          last=True, fct=fct, fcb=fcb)

    out = _sc_decode(scores.reshape(-1), ls, ld)
    return out.reshape(-1)[:E_LBL].reshape(E_LBL, 1)


# speedup vs baseline: 10.2627x; 1.0482x over previous
import functools

import jax
import jax.numpy as jnp
from jax import lax
from jax.experimental import pallas as pl
from jax.experimental.pallas import tpu as pltpu
from jax.experimental.pallas import tpu_sc as plsc

N = 10000
D_IN = 128
D_HID = 128
D_OUT = 64
NREL = 4
E = 320000
E_LBL = 100000

NC, NS, L = 2, 16, 16
NW = NC * NS
CW = 128
CH = 80
WID = 16
NPASS = D_IN // WID
E_PAD = NW * CH * CW
ROWS = 40960
RPW = ROWS // NS
LBL_PW = 3136
E_LBL_PAD = NW * LBL_PW

_MESH = dict(core_axis_name="c", subcore_axis_name="s",
             num_cores=NC, num_subcores=NS)


def _fill_zeros(ref, nrows, width):
    z16 = jnp.zeros((16,), jnp.float32)

    def body(i, _):
        for m in range(width // 16):
            ref[i, pl.ds(m * 16, 16)] = z16
        return 0

    lax.fori_loop(0, nrows, body, 0, unroll=False)


def _sc_counts_body(dst3, typ3, kidx_out, cnt_out,
                    kidxv, dstv, typv, onesb, zbufb, sem, acc_cnt):
    cid = lax.axis_index("c")
    sid = lax.axis_index("s")
    wid = cid * NS + sid
    r0 = sid * RPW

    pltpu.sync_copy(dst3.at[wid], dstv)
    pltpu.sync_copy(typ3.at[wid], typv)

    def p1(j, _):
        for m in range(CW // 16):
            d = dstv[j, pl.ds(m * 16, 16)]
            t = typv[j, pl.ds(m * 16, 16)]
            kidxv[j, pl.ds(m * 16, 16)] = t * N + d
        return 0

    lax.fori_loop(0, CH, p1, 0, unroll=False)
    pltpu.sync_copy(kidxv, kidx_out.at[wid])

    def fill_ones(i, _):
        onesb[i] = jnp.ones((16,), jnp.float32)
        return 0

    lax.fori_loop(0, CW, fill_ones, 0, unroll=False)
    _fill_zeros(zbufb, 512, 16)
    for q in range(RPW // 512):
        pltpu.sync_copy(zbufb, acc_cnt.at[pl.ds(r0 + q * 512, 512)])
    plsc.subcore_barrier()

    def p2(j, _):
        pltpu.async_copy(onesb, acc_cnt.at[kidxv.at[j]], sem, add=True)
        return 0

    lax.fori_loop(0, CH, p2, 0, unroll=False)

    def p2d(j, _):
        pltpu.make_async_copy(onesb, acc_cnt.at[kidxv.at[j]], sem).wait()
        return 0

    lax.fori_loop(0, CH, p2d, 0, unroll=False)
    plsc.subcore_barrier()
    pltpu.sync_copy(acc_cnt.at[pl.ds(r0, RPW)],
                    cnt_out.at[cid, pl.ds(r0, RPW)])


def _sc_counts(dst3, typ3):
    return pl.kernel(
        _sc_counts_body,
        out_type=[
            jax.ShapeDtypeStruct((NW, CH, CW), jnp.int32),
            jax.ShapeDtypeStruct((NC, ROWS, 16), jnp.float32),
        ],
        mesh=plsc.VectorSubcoreMesh(**_MESH),
        compiler_params=pltpu.CompilerParams(use_tc_tiling_on_sc=False),
        scratch_types=[
            pltpu.VMEM((CH, CW), jnp.int32),
            pltpu.VMEM((CH, CW), jnp.int32),
            pltpu.VMEM((CH, CW), jnp.int32),
            pltpu.VMEM((CW, 16), jnp.float32),
            pltpu.VMEM((512, 16), jnp.float32),
            pltpu.SemaphoreType.DMA,
            pltpu.VMEM_SHARED((ROWS, 16), jnp.float32),
        ],
    )(dst3, typ3)


CH2 = E_PAD // NS // CW


def _sc_layer_body(*args):
    tables = list(args[:NPASS])
    (src2, kidx2, s_out, srcv, kidxv, zbufa, gbuf, gsem, ssem, acc) = args[NPASS:]
    cid = lax.axis_index("c")
    sid = lax.axis_index("s")
    r0 = sid * RPW

    pltpu.sync_copy(src2.at[sid], srcv)
    pltpu.sync_copy(kidx2.at[sid], kidxv)
    _fill_zeros(zbufa, RPW // 2, WID)

    NB, G = 8, 6
    for g, tc in enumerate(tables):

        @pl.when(cid == g // (NPASS // NC))
        def _pass():
            pltpu.sync_copy(zbufa, acc.at[pl.ds(r0, RPW // 2)])
            pltpu.sync_copy(zbufa, acc.at[pl.ds(r0 + RPW // 2, RPW // 2)])
            plsc.subcore_barrier()

            for b in range(G):
                pltpu.async_copy(tc.at[srcv.at[b]], gbuf.at[b], gsem)

            def p3(i8, _):
                for b in range(NB):
                    j = i8 * NB + b
                    jn = j + G

                    @pl.when(j >= G)
                    def _():
                        pltpu.make_async_copy(
                            gbuf.at[(b + G) % NB],
                            acc.at[kidxv.at[j - G]], ssem).wait()

                    @pl.when(jn < CH2)
                    def _():
                        pltpu.async_copy(tc.at[srcv.at[jn]],
                                         gbuf.at[(b + G) % NB], gsem)

                    pltpu.make_async_copy(tc.at[srcv.at[j]],
                                          gbuf.at[b], gsem).wait()
                    pltpu.async_copy(gbuf.at[b], acc.at[kidxv.at[j]], ssem,
                                     add=True)
                return 0

            lax.fori_loop(0, CH2 // NB, p3, 0, unroll=False)
            for t in range(G):
                pltpu.make_async_copy(gbuf.at[t],
                                      acc.at[kidxv.at[CH2 - G + t]],
                                      ssem).wait()
            plsc.subcore_barrier()
            pltpu.sync_copy(acc.at[pl.ds(r0, RPW)],
                            s_out.at[pl.ds(r0, RPW), pl.ds(g * WID, WID)])
            plsc.subcore_barrier()


def _sc_layer(tables, src2, kidx2):
    return pl.kernel(
        _sc_layer_body,
        out_type=[
            jax.ShapeDtypeStruct((ROWS, D_IN), jnp.float32),
        ],
        mesh=plsc.VectorSubcoreMesh(**_MESH),
        compiler_params=pltpu.CompilerParams(use_tc_tiling_on_sc=False),
        scratch_types=[
            pltpu.VMEM((CH2, CW), jnp.int32),
            pltpu.VMEM((CH2, CW), jnp.int32),
            pltpu.VMEM((RPW // 2, WID), jnp.float32),
            pltpu.VMEM((8, CW, WID), jnp.float32),
            pltpu.SemaphoreType.DMA,
            pltpu.SemaphoreType.DMA,
            pltpu.VMEM_SHARED((ROWS, WID), jnp.float32),
        ],
    )(*tables, src2, kidx2)[0]


def _sc_decode_body(scores, ls, ld, out, sc_v, lsv, ldv, ob):
    cid = lax.axis_index("c")
    sid = lax.axis_index("s")
    wid = cid * NS + sid
    pltpu.sync_copy(scores, sc_v)
    pltpu.sync_copy(ls.at[wid], lsv)
    pltpu.sync_copy(ld.at[wid], ldv)

    def body(i, _):
        s = lsv[pl.ds(i * 16, 16)]
        d = ldv[pl.ds(i * 16, 16)]
        a = plsc.load_gather(sc_v, [s * 2])
        b = plsc.load_gather(sc_v, [d * 2 + 1])
        ob[pl.ds(i * 16, 16)] = a + b
        return 0

    lax.fori_loop(0, LBL_PW // 16, body, 0, unroll=False)
    pltpu.sync_copy(ob, out.at[wid])


def _sc_decode(scores, ls, ld):
    return pl.kernel(
        _sc_decode_body,
        out_type=[jax.ShapeDtypeStruct((NW, LBL_PW), jnp.float32)],
        mesh=plsc.VectorSubcoreMesh(**_MESH),
        compiler_params=pltpu.CompilerParams(use_tc_tiling_on_sc=False,
                                             needs_layout_passes=False),
        scratch_types=[
            pltpu.VMEM((2 * N,), jnp.float32),
            pltpu.VMEM((LBL_PW,), jnp.int32),
            pltpu.VMEM((LBL_PW,), jnp.int32),
            pltpu.VMEM((LBL_PW,), jnp.float32),
        ],
    )(scores, ls, ld)[0]


_BLK = 1000
_NBLK = N // _BLK


def _proj_body(x0_ref, x1_ref, w0_ref, b0_ref, w1_ref, b1_ref,
               xc_ref, *chunk_refs):
    i = pl.program_id(0)
    y0 = jnp.dot(x0_ref[...], w0_ref[...],
                 preferred_element_type=jnp.float32) + b0_ref[...]
    y1 = jnp.dot(x1_ref[...], w1_ref[...],
                 preferred_element_type=jnp.float32) + b1_ref[...]
    y = jnp.where(i < _NBLK // 2, y0, y1)
    xc_ref[...] = y
    for c, ref in enumerate(chunk_refs):
        ref[...] = y[:, c * WID:(c + 1) * WID]


def _proj(x0, x1, w0, b0, w1, b1):
    half = _NBLK // 2
    chunk_spec = pl.BlockSpec((_BLK, WID), lambda i: (i, 0))
    return pl.pallas_call(
        _proj_body,
        grid=(_NBLK,),
        in_specs=[
            pl.BlockSpec((_BLK, D_IN), lambda i: (jnp.minimum(i, half - 1), 0)),
            pl.BlockSpec((_BLK, D_IN), lambda i: (jnp.maximum(i - half, 0), 0)),
            pl.BlockSpec((D_IN, D_IN), lambda i: (0, 0)),
            pl.BlockSpec((1, D_IN), lambda i: (0, 0)),
            pl.BlockSpec((D_IN, D_IN), lambda i: (0, 0)),
            pl.BlockSpec((1, D_IN), lambda i: (0, 0)),
        ],
        out_specs=[
            pl.BlockSpec((_BLK, D_IN), lambda i: (i, 0)),
        ] + [chunk_spec] * NPASS,
        out_shape=[
            jax.ShapeDtypeStruct((N, D_IN), jnp.float32),
        ] + [jax.ShapeDtypeStruct((N, WID), jnp.float32)] * NPASS,
    )(x0, x1, w0, b0, w1, b1)


def _wbuild_body(c1_ref, b1_ref, c2_ref, b2_ref, w1_ref, w2_ref):
    w1_ref[...] = jnp.dot(c1_ref[...], b1_ref[...],
                          preferred_element_type=jnp.float32)
    w2_ref[...] = jnp.dot(c2_ref[...], b2_ref[...],
                          preferred_element_type=jnp.float32)


def _wbuild(comp1, b1f, comp2, b2f):
    return pl.pallas_call(
        _wbuild_body,
        out_shape=[
            jax.ShapeDtypeStruct((NREL, D_IN * D_HID), jnp.float32),
            jax.ShapeDtypeStruct((NREL, D_HID * D_OUT), jnp.float32),
        ],
    )(comp1, b1f, comp2, b2f)


def _layer_tc_body(dout, last, x_ref, s0, s1, s2, s3, c0, c1, c2, c3,
                   w_ref, root_ref, bias_ref, *out_refs):
    acc = jnp.dot(x_ref[...], root_ref[...],
                  preferred_element_type=jnp.float32) + bias_ref[...]
    for r, (s_ref, c_ref) in enumerate(zip((s0, s1, s2, s3),
                                           (c0, c1, c2, c3))):
        h = s_ref[...]
        cnt = c_ref[0, :, 0] + c_ref[1, :, 0]
        inv = 1.0 / jnp.maximum(cnt, 1.0)
        h = h * inv[:, None]
        acc = acc + jnp.dot(h, w_ref[r], preferred_element_type=jnp.float32)
    if not last:
        y = jnp.maximum(acc, 0.0)
        out_refs[0][...] = y
        for c in range(NPASS):
            out_refs[1 + c][...] = y[:, c * WID:(c + 1) * WID]
    else:
        fct_ref, fcb_ref, scores_ref = out_refs
        scores = jnp.dot(acc, fct_ref[...],
                         preferred_element_type=jnp.float32) + fcb_ref[...]
        scores_ref[...] = scores


def _layer_tc(x, s, cnt, w, root, bias, last, fct=None, fcb=None):
    dout = D_OUT if last else D_HID

    def s_spec(r):
        return pl.BlockSpec((_BLK, D_IN),
                            lambda i, r=r: (r * _NBLK + i, 0))

    def c_spec(r):
        return pl.BlockSpec((NC, _BLK, 16),
                            lambda i, r=r: (0, r * _NBLK + i, 0))

    in_specs = [pl.BlockSpec((_BLK, x.shape[1]), lambda i: (i, 0))]
    in_specs += [s_spec(r) for r in range(NREL)]
    in_specs += [c_spec(r) for r in range(NREL)]
    in_specs += [
        pl.BlockSpec((NREL, w.shape[1], dout), lambda i: (0, 0, 0)),
        pl.BlockSpec((root.shape[0], dout), lambda i: (0, 0)),
        pl.BlockSpec((1, dout), lambda i: (0, 0)),
    ]
    args = [x, s, s, s, s, cnt, cnt, cnt, cnt, w, root, bias]
    if not last:
        out_specs = [pl.BlockSpec((_BLK, dout), lambda i: (i, 0))]
        out_specs += [pl.BlockSpec((_BLK, WID), lambda i: (i, 0))] * NPASS
        out_shape = [jax.ShapeDtypeStruct((N, dout), jnp.float32)]
        out_shape += [jax.ShapeDtypeStruct((N, WID), jnp.float32)] * NPASS
    else:
        in_specs += [
            pl.BlockSpec((D_OUT, 2), lambda i: (0, 0)),
            pl.BlockSpec((1, 2), lambda i: (0, 0)),
        ]
        args += [fct, fcb]
        out_specs = [pl.BlockSpec((_BLK, 2), lambda i: (i, 0))]
        out_shape = [jax.ShapeDtypeStruct((N, 2), jnp.float32)]
    return pl.pallas_call(
        functools.partial(_layer_tc_body, dout, last),
        grid=(_NBLK,),
        in_specs=in_specs,
        out_specs=out_specs,
        out_shape=out_shape,
    )(*args)


def kernel(x0, x1, edge_index, edge_type, edge_label_index,
           lin0_w, lin0_b, lin1_w, lin1_b,
           bases1, comp1, root1, bias1,
           bases2, comp2, root2, bias2,
           fc_w, fc_b):
    f32 = jnp.float32
    src = edge_index[0]
    dst = edge_index[1]
    pad = E_PAD - E
    src3 = jnp.pad(src, (0, pad)).reshape(NW, CH, CW)
    dst3 = jnp.pad(dst, (0, pad)).reshape(NW, CH, CW)
    typ3 = jnp.pad(edge_type, (0, pad), constant_values=NREL).reshape(NW, CH, CW)
    trash = (jnp.arange(pad, dtype=jnp.int32) % (ROWS - NREL * N))
    dst3 = dst3.reshape(-1).at[E:].set(trash).reshape(NW, CH, CW)
    ls = jnp.pad(edge_label_index[0], (0, E_LBL_PAD - E_LBL)).reshape(NW, LBL_PW)
    ld = jnp.pad(edge_label_index[1], (0, E_LBL_PAD - E_LBL)).reshape(NW, LBL_PW)

    xcat, *xchunks = _proj(
        x0, x1, lin0_w, lin0_b.reshape(1, -1), lin1_w, lin1_b.reshape(1, -1))
    w1f, w2f = _wbuild(comp1, bases1.reshape(30, -1), comp2,
                       bases2.reshape(30, -1))
    w1 = w1f.reshape(NREL, D_IN, D_HID)
    w2 = w2f.reshape(NREL, D_HID, D_OUT)

    kidx3, cnt = _sc_counts(dst3, typ3)
    src2 = src3.reshape(NS, CH2, CW)
    kidx2 = kidx3.reshape(NS, CH2, CW)
    s1 = _sc_layer(xchunks, src2, kidx2)
    z, *zchunks = _layer_tc(
        xcat, s1, cnt, w1, root1, bias1.reshape(1, -1), last=False)
    s2 = _sc_layer(zchunks, src2, kidx2)

    fct = jnp.stack([fc_w[:D_OUT, 0], fc_w[D_OUT:, 0]], axis=1)
    fcb = jnp.concatenate([fc_b, jnp.zeros((1,), f32)]).reshape(1, 2)
    (scores,) = _layer_tc(z, s2, cnt, w2, root2, bias2.reshape(1, -1),
                          last=True, fct=fct, fcb=fcb)

    out = _sc_decode(scores.reshape(-1), ls, ld)
    return out.reshape(-1)[:E_LBL].reshape(E_LBL, 1)
